# Initial kernel scaffold; baseline (speedup 1.0000x reference)
import jax
import jax.numpy as jnp
from jax.experimental import pallas as pl


def kernel(idxes, table, beg_end):
    raise NotImplementedError("write your pallas kernel here")


# speedup vs baseline: 1.4530x; 1.4530x over previous
import functools

import jax
import jax.numpy as jnp
from jax import lax
from jax.experimental import pallas as pl
from jax.experimental.pallas import tpu as pltpu
from jax.experimental.pallas import tpu_sc as plsc

TOTAL = 1000000
NUM_EMB = TOTAL - 2
PAD = NUM_EMB - 1
START = NUM_EMB
DIM = 32
BATCH = 4096
HIST = 200

B = BATCH * HIST
NC, NS, L = 2, 16, 16
NW = NC * NS
B_PER_W = B // NW
CHUNK = 1024
GPC = CHUNK // 128
NCHUNK = B_PER_W // CHUNK


def _body(idx_hbm, table_hbm, be_hbm, out_hbm,
          idx_raw, idx_safe, rows_v, be_v, gsem):
    c = lax.axis_index("c")
    s = lax.axis_index("s")
    wid = s * NC + c
    base_w = wid * B_PER_W

    pltpu.sync_copy(be_hbm, be_v)
    lane = lax.iota(jnp.int32, L)

    def chunk_body(ci, carry):
        base = base_w + ci * CHUNK
        pltpu.sync_copy(idx_hbm.at[pl.ds(base, CHUNK)], idx_raw)

        mx = jnp.zeros((L,), jnp.int32)
        for j in range(CHUNK // L):
            v = idx_raw[pl.ds(j * L, L)]
            mx = jnp.maximum(mx, v)
            idx_safe[j // 8, pl.ds((j % 8) * L, L)] = jnp.minimum(v, PAD)

        copies = [
            pltpu.async_copy(
                table_hbm.at[idx_safe.at[g]],
                rows_v.at[pl.ds(g * 128, 128)], gsem)
            for g in range(GPC)
        ]
        for cp in copies:
            cp.wait()

        has_special = plsc.all_reduce_population_count(mx >= START)[0] > 0

        @pl.when(has_special)
        def _fixup():
            def grp_body(g, carry2):
                v = idx_raw[pl.ds(g * L, L)]
                mask = v >= START
                g_has = plsc.all_reduce_population_count(mask)[0] > 0

                @pl.when(g_has)
                def _overwrite():
                    sel = jnp.clip(v - START, 0, 1)
                    rows = g * L + lane
                    for col in range(DIM):
                        colv = jnp.full((L,), col, jnp.int32)
                        repl = plsc.load_gather(be_v, [sel, colv], mask=mask)
                        plsc.store_scatter(rows_v, [rows, colv], repl,
                                           mask=mask)
                return carry2

            lax.fori_loop(0, CHUNK // L, grp_body, 0)

        pltpu.sync_copy(rows_v, out_hbm.at[pl.ds(base, CHUNK)])
        return carry

    lax.fori_loop(0, NCHUNK, chunk_body, 0)


@jax.jit
def _run(idx_flat, table, beg_end):
    f = functools.partial(
        pl.kernel,
        mesh=plsc.VectorSubcoreMesh(core_axis_name="c", subcore_axis_name="s"),
        out_type=jax.ShapeDtypeStruct((B, DIM), jnp.float32),
        scratch_types=[
            pltpu.VMEM((CHUNK,), jnp.int32),
            pltpu.VMEM((GPC, 128), jnp.int32),
            pltpu.VMEM((CHUNK, DIM), jnp.float32),
            pltpu.VMEM((2, DIM), jnp.float32),
            pltpu.SemaphoreType.DMA,
        ],
        compiler_params=pltpu.CompilerParams(
            needs_layout_passes=False, use_tc_tiling_on_sc=False),
    )(_body)
    return f(idx_flat, table, beg_end)


def kernel(idxes, table, beg_end):
    idx_flat = idxes.reshape(B)
    out = _run(idx_flat, table, beg_end)
    return out.reshape(BATCH, HIST, DIM)


# speedup vs baseline: 1.4905x; 1.0258x over previous
import functools

import jax
import jax.numpy as jnp
from jax import lax
from jax.experimental import pallas as pl
from jax.experimental.pallas import tpu as pltpu
from jax.experimental.pallas import tpu_sc as plsc

TOTAL = 1000000
NUM_EMB = TOTAL - 2
PAD = NUM_EMB - 1
START = NUM_EMB
DIM = 32
BATCH = 4096
HIST = 200

B = BATCH * HIST
NC, NS, L = 2, 16, 16
NW = NC * NS
B_PER_W = B // NW
CHUNK = 1280
GPC = CHUNK // 128
NCHUNK = B_PER_W // CHUNK


def _body(idx_hbm, table_hbm, be_hbm, out_hbm,
          ir0, ir1, is0, is1, rv0, rv1, be_v,
          isem0, isem1, gsem0, gsem1, osem0, osem1):
    c = lax.axis_index("c")
    s = lax.axis_index("s")
    wid = s * NC + c
    base_w = wid * B_PER_W

    pltpu.sync_copy(be_hbm, be_v)
    lane = lax.iota(jnp.int32, L)

    IR = (ir0, ir1)
    IS = (is0, is1)
    RV = (rv0, rv1)
    ISEM = (isem0, isem1)
    GSEM = (gsem0, gsem1)
    OSEM = (osem0, osem1)

    def idx_copy(ci, b):
        return pltpu.make_async_copy(
            idx_hbm.at[pl.ds(base_w + ci * CHUNK, CHUNK)], IR[b], ISEM[b])

    def out_copy(ci, b):
        return pltpu.make_async_copy(
            RV[b], out_hbm.at[pl.ds(base_w + ci * CHUNK, CHUNK)], OSEM[b])

    def pass1(b):
        mx = jnp.zeros((L,), jnp.int32)
        for j in range(CHUNK // L):
            v = IR[b][pl.ds(j * L, L)]
            mx = jnp.maximum(mx, v)
            IS[b][j // 8, pl.ds((j % 8) * L, L)] = jnp.minimum(v, PAD)
        return mx

    def fixup(b, mx):
        has_special = plsc.all_reduce_population_count(mx >= START)[0] > 0

        @pl.when(has_special)
        def _fix():
            def grp_body(g, carry):
                v = IR[b][pl.ds(g * L, L)]
                mask = v >= START
                g_has = plsc.all_reduce_population_count(mask)[0] > 0

                @pl.when(g_has)
                def _overwrite():
                    sel = jnp.clip(v - START, 0, 1)
                    rows = g * L + lane
                    for col in range(DIM):
                        colv = jnp.full((L,), col, jnp.int32)
                        repl = plsc.load_gather(be_v, [sel, colv], mask=mask)
                        plsc.store_scatter(RV[b], [rows, colv], repl,
                                           mask=mask)
                return carry

            lax.fori_loop(0, CHUNK // L, grp_body, 0)

    def stage(ci, b, wait_prev_out, fire_next_idx):
        idx_copy(ci, b).wait()
        mx = pass1(b)
        if wait_prev_out:
            out_copy(ci, b).wait()
        gathers = [
            pltpu.async_copy(
                table_hbm.at[IS[b].at[g]],
                RV[b].at[pl.ds(g * 128, 128)], GSEM[b])
            for g in range(GPC)
        ]
        if fire_next_idx:
            idx_copy(ci + 1, 1 - b).start()
        for cp in gathers:
            cp.wait()
        fixup(b, mx)
        out_copy(ci, b).start()

    idx_copy(0, 0).start()
    stage(0, 0, False, True)
    stage(1, 1, False, True)

    def pair(g, carry):
        ci = 2 + 2 * g
        stage(ci, 0, True, True)
        stage(ci + 1, 1, True, True)
        return carry

    lax.fori_loop(0, (NCHUNK - 4) // 2, pair, 0)
    stage(NCHUNK - 2, 0, True, True)
    stage(NCHUNK - 1, 1, True, False)
    out_copy(NCHUNK - 2, 0).wait()
    out_copy(NCHUNK - 1, 1).wait()


@jax.jit
def _run(idx_flat, table, beg_end):
    f = functools.partial(
        pl.kernel,
        mesh=plsc.VectorSubcoreMesh(core_axis_name="c", subcore_axis_name="s"),
        out_type=jax.ShapeDtypeStruct((B, DIM), jnp.float32),
        scratch_types=[
            pltpu.VMEM((CHUNK,), jnp.int32),
            pltpu.VMEM((CHUNK,), jnp.int32),
            pltpu.VMEM((GPC, 128), jnp.int32),
            pltpu.VMEM((GPC, 128), jnp.int32),
            pltpu.VMEM((CHUNK, DIM), jnp.float32),
            pltpu.VMEM((CHUNK, DIM), jnp.float32),
            pltpu.VMEM((2, DIM), jnp.float32),
            pltpu.SemaphoreType.DMA,
            pltpu.SemaphoreType.DMA,
            pltpu.SemaphoreType.DMA,
            pltpu.SemaphoreType.DMA,
            pltpu.SemaphoreType.DMA,
            pltpu.SemaphoreType.DMA,
        ],
        compiler_params=pltpu.CompilerParams(
            needs_layout_passes=False, use_tc_tiling_on_sc=False),
    )(_body)
    return f(idx_flat, table, beg_end)


def kernel(idxes, table, beg_end):
    idx_flat = idxes.reshape(B)
    out = _run(idx_flat, table, beg_end)
    return out.reshape(BATCH, HIST, DIM)


# speedup vs baseline: 1.4907x; 1.0001x over previous
import functools

import jax
import jax.numpy as jnp
from jax import lax
from jax.experimental import pallas as pl
from jax.experimental.pallas import tpu as pltpu
from jax.experimental.pallas import tpu_sc as plsc

TOTAL = 1000000
NUM_EMB = TOTAL - 2
PAD = NUM_EMB - 1
START = NUM_EMB
DIM = 32
BATCH = 4096
HIST = 200

B = BATCH * HIST
NC, NS, L = 2, 16, 16
NW = NC * NS
B_PER_W = B // NW
CHUNK = 1280
GPC = CHUNK // 128
NCHUNK = B_PER_W // CHUNK


def _body(idx_hbm, table_hbm, be_hbm, out_hbm,
          ir0, ir1, is0, is1, rv0, rv1, be_v,
          isem0, isem1, gsem0, gsem1, osem0, osem1):
    c = lax.axis_index("c")
    s = lax.axis_index("s")
    wid = s * NC + c
    base_w = wid * B_PER_W

    pltpu.sync_copy(be_hbm, be_v)
    lane = lax.iota(jnp.int32, L)

    IR = (ir0, ir1)
    IS = (is0, is1)
    RV = (rv0, rv1)
    ISEM = (isem0, isem1)
    GSEM = (gsem0, gsem1)
    OSEM = (osem0, osem1)

    def idx_copy(ci, b):
        return pltpu.make_async_copy(
            idx_hbm.at[pl.ds(base_w + ci * CHUNK, CHUNK)], IR[b], ISEM[b])

    def out_copy(ci, b):
        return pltpu.make_async_copy(
            RV[b], out_hbm.at[pl.ds(base_w + ci * CHUNK, CHUNK)], OSEM[b])

    def pass1(b):
        mx = jnp.zeros((L,), jnp.int32)
        for j in range(CHUNK // L):
            v = IR[b][pl.ds(j * L, L)]
            mx = jnp.maximum(mx, v)
            IS[b][pl.ds(j * L, L)] = jnp.minimum(v, PAD)
        return mx

    def fixup(b, mx):
        has_special = plsc.all_reduce_population_count(mx >= START)[0] > 0

        @pl.when(has_special)
        def _fix():
            def grp_body(g, carry):
                v = IR[b][pl.ds(g * L, L)]
                mask = v >= START
                g_has = plsc.all_reduce_population_count(mask)[0] > 0

                @pl.when(g_has)
                def _overwrite():
                    sel = jnp.clip(v - START, 0, 1)
                    rows = g * L + lane
                    for col in range(DIM):
                        colv = jnp.full((L,), col, jnp.int32)
                        repl = plsc.load_gather(be_v, [sel, colv], mask=mask)
                        plsc.store_scatter(RV[b], [rows, colv], repl,
                                           mask=mask)
                return carry

            lax.fori_loop(0, CHUNK // L, grp_body, 0)

    def stage(ci, b, wait_prev_out, fire_next_idx):
        idx_copy(ci, b).wait()
        mx = pass1(b)
        if wait_prev_out:
            out_copy(ci, b).wait()
        gather = pltpu.async_copy(table_hbm.at[IS[b]], RV[b], GSEM[b])
        if fire_next_idx:
            idx_copy(ci + 1, 1 - b).start()
        gather.wait()
        fixup(b, mx)
        out_copy(ci, b).start()

    idx_copy(0, 0).start()
    stage(0, 0, False, True)
    stage(1, 1, False, True)

    def pair(g, carry):
        ci = 2 + 2 * g
        stage(ci, 0, True, True)
        stage(ci + 1, 1, True, True)
        return carry

    lax.fori_loop(0, (NCHUNK - 4) // 2, pair, 0)
    stage(NCHUNK - 2, 0, True, True)
    stage(NCHUNK - 1, 1, True, False)
    out_copy(NCHUNK - 2, 0).wait()
    out_copy(NCHUNK - 1, 1).wait()


@jax.jit
def _run(idx_flat, table, beg_end):
    f = functools.partial(
        pl.kernel,
        mesh=plsc.VectorSubcoreMesh(core_axis_name="c", subcore_axis_name="s"),
        out_type=jax.ShapeDtypeStruct((B, DIM), jnp.float32),
        scratch_types=[
            pltpu.VMEM((CHUNK,), jnp.int32),
            pltpu.VMEM((CHUNK,), jnp.int32),
            pltpu.VMEM((CHUNK,), jnp.int32),
            pltpu.VMEM((CHUNK,), jnp.int32),
            pltpu.VMEM((CHUNK, DIM), jnp.float32),
            pltpu.VMEM((CHUNK, DIM), jnp.float32),
            pltpu.VMEM((2, DIM), jnp.float32),
            pltpu.SemaphoreType.DMA,
            pltpu.SemaphoreType.DMA,
            pltpu.SemaphoreType.DMA,
            pltpu.SemaphoreType.DMA,
            pltpu.SemaphoreType.DMA,
            pltpu.SemaphoreType.DMA,
        ],
        compiler_params=pltpu.CompilerParams(
            needs_layout_passes=False, use_tc_tiling_on_sc=False),
    )(_body)
    return f(idx_flat, table, beg_end)


def kernel(idxes, table, beg_end):
    idx_flat = idxes.reshape(B)
    out = _run(idx_flat, table, beg_end)
    return out.reshape(BATCH, HIST, DIM)
